# baseline (speedup 1.0000x reference)
import functools

import jax
import jax.numpy as jnp
from jax.experimental import pallas as pl

N_SRC = 20000
N_QUERY = 2500
C_IN = 256
C_OUT = 512
K = 16

_PROJ_BLK = 400


def _proj_body(feats_ref, wt_ref, gamma_ref, beta_ref, out_ref):
    f = feats_ref[...]
    mean = jnp.mean(f, axis=1, keepdims=True)
    cent = f - mean
    var = jnp.mean(cent * cent, axis=1, keepdims=True)
    normed = cent * jax.lax.rsqrt(var + 1e-5) * gamma_ref[...] + beta_ref[...]
    out_ref[...] = jnp.dot(normed, wt_ref[...], preferred_element_type=jnp.float32)


def _project_all(feats, W, ln_gamma, ln_beta):
    wt = W.T
    gamma = ln_gamma.reshape(1, C_IN)
    beta = ln_beta.reshape(1, C_IN)
    grid = N_SRC // _PROJ_BLK
    return pl.pallas_call(
        _proj_body,
        grid=(grid,),
        in_specs=[
            pl.BlockSpec((_PROJ_BLK, C_IN), lambda i: (i, 0)),
            pl.BlockSpec((C_IN, C_OUT), lambda i: (0, 0)),
            pl.BlockSpec((1, C_IN), lambda i: (0, 0)),
            pl.BlockSpec((1, C_IN), lambda i: (0, 0)),
        ],
        out_specs=pl.BlockSpec((_PROJ_BLK, C_OUT), lambda i: (i, 0)),
        out_shape=jax.ShapeDtypeStruct((N_SRC, C_OUT), jnp.float32),
    )(feats, wt, gamma, beta)


def kernel(xyz, n_xyz, feats, ln_gamma, ln_beta, W):
    proj = _project_all(feats, W, ln_gamma, ln_beta)
    q2 = jnp.sum(n_xyz * n_xyz, axis=1, keepdims=True)
    s2 = jnp.sum(xyz * xyz, axis=1)[None, :]
    d2 = q2 - 2.0 * (n_xyz @ xyz.T) + s2
    _, idx = jax.lax.top_k(-d2, K)
    pooled = jnp.max(proj[idx], axis=1)
    return pooled
